# baseline (speedup 1.0000x reference)
import functools

import jax
import jax.numpy as jnp
from jax import lax
from jax.experimental import pallas as pl
from jax.experimental.pallas import tpu as pltpu
from jax.experimental.pallas import tpu_sc as plsc

N = 10000
E = 320000
D_IN = 128
D_HID = 64
D_OUT = 128

NC, NS = 2, 16
NW = NC * NS
NPAD = 10240
C = 128
NCH_TOT = 162
EPAD = NS * NCH_TOT * C
EPW = EPAD // NW
ROWS_PT = NPAD // NS

_sc_mesh = plsc.VectorSubcoreMesh(
    core_axis_name="c", subcore_axis_name="s", num_cores=NC, num_subcores=NS
)

DEG_CHUNK = 1152


@functools.partial(
    pl.kernel,
    out_type=jax.ShapeDtypeStruct((NW, NPAD), jnp.float32),
    mesh=_sc_mesh,
    scratch_types=[
        pltpu.VMEM((NPAD,), jnp.float32),
        pltpu.VMEM((DEG_CHUNK,), jnp.int32),
    ],
    compiler_params=pltpu.CompilerParams(needs_layout_passes=False),
)
def _sc_degree(dst_hbm, out_hbm, hist_v, dbuf_v):
    cid = lax.axis_index("c")
    sid = lax.axis_index("s")
    wid = sid * NC + cid

    def zero_body(j, carry):
        hist_v[pl.ds(j * 16, 16)] = jnp.zeros((16,), jnp.float32)
        return carry

    lax.fori_loop(0, NPAD // 16, zero_body, None)

    base = wid * EPW
    ones = jnp.ones((16,), jnp.float32)

    def chunk_body(i, carry):
        pltpu.sync_copy(dst_hbm.at[pl.ds(base + i * DEG_CHUNK, DEG_CHUNK)], dbuf_v)

        def inner(k, c2):
            idx = dbuf_v[pl.ds(k * 16, 16)]
            plsc.addupdate_scatter(hist_v, [idx], ones)
            return c2

        lax.fori_loop(0, DEG_CHUNK // 16, inner, None)
        return carry

    lax.fori_loop(0, EPW // DEG_CHUNK, chunk_body, None)
    pltpu.sync_copy(hist_v, out_hbm.at[wid])


K = 3
CH_SLOW = 36
CH_FAST = NCH_TOT - CH_SLOW
SLOW_CID = 1
NG_FAST = CH_FAST // K
NG_SLOW = CH_SLOW // K
CAP = CH_FAST


@functools.partial(
    pl.kernel,
    out_type=jax.ShapeDtypeStruct((NC, NPAD, D_HID), jnp.float32),
    mesh=_sc_mesh,
    scratch_types=[
        pltpu.VMEM((CAP, C), jnp.int32),
        pltpu.VMEM((CAP, C), jnp.int32),
        pltpu.VMEM((2 * K, C, D_HID), jnp.float32),
        pltpu.VMEM_SHARED((NPAD, D_HID), jnp.float32),
        pltpu.SemaphoreType.DMA,
        pltpu.SemaphoreType.DMA,
        pltpu.SemaphoreType.DMA,
    ],
    compiler_params=pltpu.CompilerParams(
        needs_layout_passes=False, use_tc_tiling_on_sc=False
    ),
)
def _sc_aggregate(y_hbm, src_hbm, dst_hbm, out_hbm, srci, dsti, rows, acc_sh,
                  sem_a, sem_b, sem_s):
    cid = lax.axis_index("c")
    sid = lax.axis_index("s")

    ro = jnp.where(cid == SLOW_CID, NCH_TOT - CAP, 0)
    pltpu.sync_copy(src_hbm.at[sid, pl.ds(ro, CAP)], srci)
    pltpu.sync_copy(dst_hbm.at[sid, pl.ds(ro, CAP)], dsti)

    r0 = sid * ROWS_PT
    pltpu.sync_copy(y_hbm.at[pl.ds(r0, ROWS_PT)], acc_sh.at[pl.ds(r0, ROWS_PT)])
    plsc.subcore_barrier()

    base = jnp.where(cid == SLOW_CID, CAP - CH_SLOW, 0)
    ng = jnp.where(cid == SLOW_CID, NG_SLOW, NG_FAST)

    def fire(g, bufset, sem):
        for b in range(K):
            pltpu.async_copy(y_hbm.at[srci.at[base + g * K + b]],
                             rows.at[bufset * K + b], sem)

    def drain_scatter(g, bufset, sem):
        for b in range(K):
            pltpu.make_async_copy(y_hbm.at[pl.ds(0, C)],
                                  rows.at[bufset * K + b], sem).wait()
        for b in range(K):
            pltpu.async_copy(rows.at[bufset * K + b],
                             acc_sh.at[dsti.at[base + g * K + b]], sem_s, add=True)
        for b in range(K):
            pltpu.make_async_copy(y_hbm.at[pl.ds(0, C)],
                                  rows.at[bufset * K + b], sem_s).wait()

    fire(0, 0, sem_a)
    fire(1, 1, sem_b)

    def body(t, carry):
        g = 2 * t
        drain_scatter(g, 0, sem_a)
        fire(g + 2, 0, sem_a)
        drain_scatter(g + 1, 1, sem_b)
        fire(g + 3, 1, sem_b)
        return carry

    lax.fori_loop(0, ng // 2 - 1, body, None)
    drain_scatter(ng - 2, 0, sem_a)
    drain_scatter(ng - 1, 1, sem_b)

    plsc.subcore_barrier()
    pltpu.sync_copy(acc_sh.at[pl.ds(r0, ROWS_PT)], out_hbm.at[cid, pl.ds(r0, ROWS_PT)])


ROW_BLK = 2048


def _tc_prep_body(hist_ref, x_ref, w_ref, y_ref, dinv_ref):
    deg = jnp.sum(hist_ref[...], axis=0) + 1.0
    dinv = lax.rsqrt(deg)
    xw = jnp.dot(x_ref[...], w_ref[...], preferred_element_type=jnp.float32)
    y_ref[...] = xw * dinv[:, None]
    dinv_ref[...] = dinv[:, None]


def _tc_prep(hist, xp, W1):
    grid = NPAD // ROW_BLK
    return pl.pallas_call(
        _tc_prep_body,
        grid=(grid,),
        in_specs=[
            pl.BlockSpec((NW, ROW_BLK), lambda i: (0, i)),
            pl.BlockSpec((ROW_BLK, D_IN), lambda i: (i, 0)),
            pl.BlockSpec((D_IN, D_HID), lambda i: (0, 0)),
        ],
        out_specs=[
            pl.BlockSpec((ROW_BLK, D_HID), lambda i: (i, 0)),
            pl.BlockSpec((ROW_BLK, 1), lambda i: (i, 0)),
        ],
        out_shape=[
            jax.ShapeDtypeStruct((NPAD, D_HID), jnp.float32),
            jax.ShapeDtypeStruct((NPAD, 1), jnp.float32),
        ],
    )(hist, xp, W1)


def _tc_mid_body(p_ref, y_ref, dinv_ref, w_ref, b_ref, out_ref):
    dinv = dinv_ref[...]
    s = p_ref[0] + p_ref[1] - y_ref[...]
    h = jnp.maximum(dinv * s + b_ref[...], 0.0)
    out_ref[...] = jnp.dot(h, w_ref[...], preferred_element_type=jnp.float32) * dinv


def _tc_mid(p, y, dinv, W, b):
    grid = NPAD // ROW_BLK
    return pl.pallas_call(
        _tc_mid_body,
        grid=(grid,),
        in_specs=[
            pl.BlockSpec((NC, ROW_BLK, D_HID), lambda i: (0, i, 0)),
            pl.BlockSpec((ROW_BLK, D_HID), lambda i: (i, 0)),
            pl.BlockSpec((ROW_BLK, 1), lambda i: (i, 0)),
            pl.BlockSpec((D_HID, D_HID), lambda i: (0, 0)),
            pl.BlockSpec((1, D_HID), lambda i: (0, 0)),
        ],
        out_specs=pl.BlockSpec((ROW_BLK, D_HID), lambda i: (i, 0)),
        out_shape=jax.ShapeDtypeStruct((NPAD, D_HID), jnp.float32),
    )(p, y, dinv, W, b)


def _tc_mid2_body(p_ref, y_ref, dinv_ref, b_ref, out_ref):
    dinv = dinv_ref[...]
    s = p_ref[0] + p_ref[1] - y_ref[...]
    out_ref[...] = dinv * jnp.maximum(dinv * s + b_ref[...], 0.0)


def _tc_mid2(p, y, dinv, b):
    grid = NPAD // ROW_BLK
    return pl.pallas_call(
        _tc_mid2_body,
        grid=(grid,),
        in_specs=[
            pl.BlockSpec((NC, ROW_BLK, D_HID), lambda i: (0, i, 0)),
            pl.BlockSpec((ROW_BLK, D_HID), lambda i: (i, 0)),
            pl.BlockSpec((ROW_BLK, 1), lambda i: (i, 0)),
            pl.BlockSpec((1, D_HID), lambda i: (0, 0)),
        ],
        out_specs=pl.BlockSpec((ROW_BLK, D_HID), lambda i: (i, 0)),
        out_shape=jax.ShapeDtypeStruct((NPAD, D_HID), jnp.float32),
    )(p, y, dinv, b)


def _tc_fin_body(p_ref, y_ref, dinv_ref, w_ref, b_ref, out_ref):
    dinv = dinv_ref[...]
    z = dinv * (p_ref[0] + p_ref[1] - y_ref[...])
    zw = jnp.dot(z, w_ref[...], preferred_element_type=jnp.float32)
    out_ref[...] = jnp.maximum(zw + b_ref[...], 0.0)


def _tc_fin(p, y, dinv, W, b):
    grid = NPAD // ROW_BLK
    return pl.pallas_call(
        _tc_fin_body,
        grid=(grid,),
        in_specs=[
            pl.BlockSpec((NC, ROW_BLK, D_HID), lambda i: (0, i, 0)),
            pl.BlockSpec((ROW_BLK, D_HID), lambda i: (i, 0)),
            pl.BlockSpec((ROW_BLK, 1), lambda i: (i, 0)),
            pl.BlockSpec((D_HID, D_OUT), lambda i: (0, 0)),
            pl.BlockSpec((1, D_OUT), lambda i: (0, 0)),
        ],
        out_specs=pl.BlockSpec((ROW_BLK, D_OUT), lambda i: (i, 0)),
        out_shape=jax.ShapeDtypeStruct((NPAD, D_OUT), jnp.float32),
    )(p, y, dinv, W, b)


def kernel(x, edge_index, W1, b1, W3, b3, W2, b2):
    pad_idx = jnp.full((EPAD - E,), N, dtype=jnp.int32)
    srcp = jnp.concatenate([edge_index[0], pad_idx])
    dstp = jnp.concatenate([edge_index[1], pad_idx])
    xp = jnp.pad(x, ((0, NPAD - N), (0, 0)))

    src3 = srcp.reshape(NS, NCH_TOT, C)
    dst3 = dstp.reshape(NS, NCH_TOT, C)

    hist = _sc_degree(dstp)
    y1, dinv = _tc_prep(hist, xp, W1)
    p1 = _sc_aggregate(y1, src3, dst3)
    y2 = _tc_mid(p1, y1, dinv, W3, b1.reshape(1, -1))
    p2 = _sc_aggregate(y2, src3, dst3)
    y3 = _tc_mid2(p2, y2, dinv, b3.reshape(1, -1))
    p3 = _sc_aggregate(y3, src3, dst3)
    out = _tc_fin(p3, y3, dinv, W2, b2.reshape(1, -1))
    return out[:N]


# speedup vs baseline: 1.0207x; 1.0207x over previous
import functools

import jax
import jax.numpy as jnp
from jax import lax
from jax.experimental import pallas as pl
from jax.experimental.pallas import tpu as pltpu
from jax.experimental.pallas import tpu_sc as plsc

N = 10000
E = 320000
D_IN = 128
D_HID = 64
D_OUT = 128

NC, NS = 2, 16
NW = NC * NS
NPAD = 10240
C = 128
NCH_TOT = 162
EPAD = NS * NCH_TOT * C
EPW = EPAD // NW
ROWS_PT = NPAD // NS

_sc_mesh = plsc.VectorSubcoreMesh(
    core_axis_name="c", subcore_axis_name="s", num_cores=NC, num_subcores=NS
)

DEG_CHUNK = 1152


@functools.partial(
    pl.kernel,
    out_type=jax.ShapeDtypeStruct((NW, NPAD), jnp.float32),
    mesh=_sc_mesh,
    scratch_types=[
        pltpu.VMEM((NPAD,), jnp.float32),
        pltpu.VMEM((DEG_CHUNK,), jnp.int32),
    ],
    compiler_params=pltpu.CompilerParams(needs_layout_passes=False),
)
def _sc_degree(dst_hbm, out_hbm, hist_v, dbuf_v):
    cid = lax.axis_index("c")
    sid = lax.axis_index("s")
    wid = sid * NC + cid

    def zero_body(j, carry):
        hist_v[pl.ds(j * 16, 16)] = jnp.zeros((16,), jnp.float32)
        return carry

    lax.fori_loop(0, NPAD // 16, zero_body, None)

    base = wid * EPW
    ones = jnp.ones((16,), jnp.float32)

    def chunk_body(i, carry):
        pltpu.sync_copy(dst_hbm.at[pl.ds(base + i * DEG_CHUNK, DEG_CHUNK)], dbuf_v)

        def inner(k, c2):
            idx = dbuf_v[pl.ds(k * 16, 16)]
            plsc.addupdate_scatter(hist_v, [idx], ones)
            return c2

        lax.fori_loop(0, DEG_CHUNK // 16, inner, None)
        return carry

    lax.fori_loop(0, EPW // DEG_CHUNK, chunk_body, None)
    pltpu.sync_copy(hist_v, out_hbm.at[wid])


K = 3
CH_SLOW = 36
CH_FAST = NCH_TOT - CH_SLOW
SLOW_CID = 0
NG_FAST = CH_FAST // K
NG_SLOW = CH_SLOW // K
CAP = CH_FAST


@functools.partial(
    pl.kernel,
    out_type=jax.ShapeDtypeStruct((NC, NPAD, D_HID), jnp.float32),
    mesh=_sc_mesh,
    scratch_types=[
        pltpu.VMEM((CAP, C), jnp.int32),
        pltpu.VMEM((CAP, C), jnp.int32),
        pltpu.VMEM((2 * K, C, D_HID), jnp.float32),
        pltpu.VMEM_SHARED((NPAD, D_HID), jnp.float32),
        pltpu.SemaphoreType.DMA,
        pltpu.SemaphoreType.DMA,
        pltpu.SemaphoreType.DMA,
    ],
    compiler_params=pltpu.CompilerParams(
        needs_layout_passes=False, use_tc_tiling_on_sc=False
    ),
)
def _sc_aggregate(y_hbm, src_hbm, dst_hbm, out_hbm, srci, dsti, rows, acc_sh,
                  sem_a, sem_b, sem_s):
    cid = lax.axis_index("c")
    sid = lax.axis_index("s")

    ro = jnp.where(cid == SLOW_CID, NCH_TOT - CAP, 0)
    pltpu.sync_copy(src_hbm.at[sid, pl.ds(ro, CAP)], srci)
    pltpu.sync_copy(dst_hbm.at[sid, pl.ds(ro, CAP)], dsti)

    r0 = sid * ROWS_PT
    pltpu.sync_copy(y_hbm.at[pl.ds(r0, ROWS_PT)], acc_sh.at[pl.ds(r0, ROWS_PT)])
    plsc.subcore_barrier()

    base = jnp.where(cid == SLOW_CID, CAP - CH_SLOW, 0)
    ng = jnp.where(cid == SLOW_CID, NG_SLOW, NG_FAST)

    def fire(g, bufset, sem):
        for b in range(K):
            pltpu.async_copy(y_hbm.at[srci.at[base + g * K + b]],
                             rows.at[bufset * K + b], sem)

    def drain_scatter(g, bufset, sem):
        for b in range(K):
            pltpu.make_async_copy(y_hbm.at[pl.ds(0, C)],
                                  rows.at[bufset * K + b], sem).wait()
        for b in range(K):
            pltpu.async_copy(rows.at[bufset * K + b],
                             acc_sh.at[dsti.at[base + g * K + b]], sem_s, add=True)
        for b in range(K):
            pltpu.make_async_copy(y_hbm.at[pl.ds(0, C)],
                                  rows.at[bufset * K + b], sem_s).wait()

    fire(0, 0, sem_a)
    fire(1, 1, sem_b)

    def body(t, carry):
        g = 2 * t
        drain_scatter(g, 0, sem_a)
        fire(g + 2, 0, sem_a)
        drain_scatter(g + 1, 1, sem_b)
        fire(g + 3, 1, sem_b)
        return carry

    lax.fori_loop(0, ng // 2 - 1, body, None)
    drain_scatter(ng - 2, 0, sem_a)
    drain_scatter(ng - 1, 1, sem_b)

    plsc.subcore_barrier()
    pltpu.sync_copy(acc_sh.at[pl.ds(r0, ROWS_PT)], out_hbm.at[cid, pl.ds(r0, ROWS_PT)])


ROW_BLK = 2048


def _tc_prep_body(hist_ref, x_ref, w_ref, y_ref, dinv_ref):
    deg = jnp.sum(hist_ref[...], axis=0) + 1.0
    dinv = lax.rsqrt(deg)
    xw = jnp.dot(x_ref[...], w_ref[...], preferred_element_type=jnp.float32)
    y_ref[...] = xw * dinv[:, None]
    dinv_ref[...] = dinv[:, None]


def _tc_prep(hist, xp, W1):
    grid = NPAD // ROW_BLK
    return pl.pallas_call(
        _tc_prep_body,
        grid=(grid,),
        in_specs=[
            pl.BlockSpec((NW, ROW_BLK), lambda i: (0, i)),
            pl.BlockSpec((ROW_BLK, D_IN), lambda i: (i, 0)),
            pl.BlockSpec((D_IN, D_HID), lambda i: (0, 0)),
        ],
        out_specs=[
            pl.BlockSpec((ROW_BLK, D_HID), lambda i: (i, 0)),
            pl.BlockSpec((ROW_BLK, 1), lambda i: (i, 0)),
        ],
        out_shape=[
            jax.ShapeDtypeStruct((NPAD, D_HID), jnp.float32),
            jax.ShapeDtypeStruct((NPAD, 1), jnp.float32),
        ],
    )(hist, xp, W1)


def _tc_mid_body(p_ref, y_ref, dinv_ref, w_ref, b_ref, out_ref):
    dinv = dinv_ref[...]
    s = p_ref[0] + p_ref[1] - y_ref[...]
    h = jnp.maximum(dinv * s + b_ref[...], 0.0)
    out_ref[...] = jnp.dot(h, w_ref[...], preferred_element_type=jnp.float32) * dinv


def _tc_mid(p, y, dinv, W, b):
    grid = NPAD // ROW_BLK
    return pl.pallas_call(
        _tc_mid_body,
        grid=(grid,),
        in_specs=[
            pl.BlockSpec((NC, ROW_BLK, D_HID), lambda i: (0, i, 0)),
            pl.BlockSpec((ROW_BLK, D_HID), lambda i: (i, 0)),
            pl.BlockSpec((ROW_BLK, 1), lambda i: (i, 0)),
            pl.BlockSpec((D_HID, D_HID), lambda i: (0, 0)),
            pl.BlockSpec((1, D_HID), lambda i: (0, 0)),
        ],
        out_specs=pl.BlockSpec((ROW_BLK, D_HID), lambda i: (i, 0)),
        out_shape=jax.ShapeDtypeStruct((NPAD, D_HID), jnp.float32),
    )(p, y, dinv, W, b)


def _tc_mid2_body(p_ref, y_ref, dinv_ref, b_ref, out_ref):
    dinv = dinv_ref[...]
    s = p_ref[0] + p_ref[1] - y_ref[...]
    out_ref[...] = dinv * jnp.maximum(dinv * s + b_ref[...], 0.0)


def _tc_mid2(p, y, dinv, b):
    grid = NPAD // ROW_BLK
    return pl.pallas_call(
        _tc_mid2_body,
        grid=(grid,),
        in_specs=[
            pl.BlockSpec((NC, ROW_BLK, D_HID), lambda i: (0, i, 0)),
            pl.BlockSpec((ROW_BLK, D_HID), lambda i: (i, 0)),
            pl.BlockSpec((ROW_BLK, 1), lambda i: (i, 0)),
            pl.BlockSpec((1, D_HID), lambda i: (0, 0)),
        ],
        out_specs=pl.BlockSpec((ROW_BLK, D_HID), lambda i: (i, 0)),
        out_shape=jax.ShapeDtypeStruct((NPAD, D_HID), jnp.float32),
    )(p, y, dinv, b)


def _tc_fin_body(p_ref, y_ref, dinv_ref, w_ref, b_ref, out_ref):
    dinv = dinv_ref[...]
    z = dinv * (p_ref[0] + p_ref[1] - y_ref[...])
    zw = jnp.dot(z, w_ref[...], preferred_element_type=jnp.float32)
    out_ref[...] = jnp.maximum(zw + b_ref[...], 0.0)


def _tc_fin(p, y, dinv, W, b):
    grid = NPAD // ROW_BLK
    return pl.pallas_call(
        _tc_fin_body,
        grid=(grid,),
        in_specs=[
            pl.BlockSpec((NC, ROW_BLK, D_HID), lambda i: (0, i, 0)),
            pl.BlockSpec((ROW_BLK, D_HID), lambda i: (i, 0)),
            pl.BlockSpec((ROW_BLK, 1), lambda i: (i, 0)),
            pl.BlockSpec((D_HID, D_OUT), lambda i: (0, 0)),
            pl.BlockSpec((1, D_OUT), lambda i: (0, 0)),
        ],
        out_specs=pl.BlockSpec((ROW_BLK, D_OUT), lambda i: (i, 0)),
        out_shape=jax.ShapeDtypeStruct((NPAD, D_OUT), jnp.float32),
    )(p, y, dinv, W, b)


def kernel(x, edge_index, W1, b1, W3, b3, W2, b2):
    pad_idx = jnp.full((EPAD - E,), N, dtype=jnp.int32)
    srcp = jnp.concatenate([edge_index[0], pad_idx])
    dstp = jnp.concatenate([edge_index[1], pad_idx])
    xp = jnp.pad(x, ((0, NPAD - N), (0, 0)))

    src3 = srcp.reshape(NS, NCH_TOT, C)
    dst3 = dstp.reshape(NS, NCH_TOT, C)

    hist = _sc_degree(dstp)
    y1, dinv = _tc_prep(hist, xp, W1)
    p1 = _sc_aggregate(y1, src3, dst3)
    y2 = _tc_mid(p1, y1, dinv, W3, b1.reshape(1, -1))
    p2 = _sc_aggregate(y2, src3, dst3)
    y3 = _tc_mid2(p2, y2, dinv, b3.reshape(1, -1))
    p3 = _sc_aggregate(y3, src3, dst3)
    out = _tc_fin(p3, y3, dinv, W2, b2.reshape(1, -1))
    return out[:N]


# speedup vs baseline: 1.2264x; 1.2016x over previous
import functools

import jax
import jax.numpy as jnp
from jax import lax
from jax.experimental import pallas as pl
from jax.experimental.pallas import tpu as pltpu
from jax.experimental.pallas import tpu_sc as plsc

N = 10000
E = 320000
D_IN = 128
D_HID = 64
D_OUT = 128

NC, NS = 2, 16
NW = NC * NS
NPAD = 10240
C = 128
NCH_TOT = 160
EPAD = NS * NCH_TOT * C
EPW = EPAD // NW
ROWS_PT = NPAD // NS

_sc_mesh = plsc.VectorSubcoreMesh(
    core_axis_name="c", subcore_axis_name="s", num_cores=NC, num_subcores=NS
)

DEG_CHUNK = 1024


@functools.partial(
    pl.kernel,
    out_type=jax.ShapeDtypeStruct((NW, NPAD), jnp.float32),
    mesh=_sc_mesh,
    scratch_types=[
        pltpu.VMEM((NPAD,), jnp.float32),
        pltpu.VMEM((DEG_CHUNK,), jnp.int32),
    ],
    compiler_params=pltpu.CompilerParams(needs_layout_passes=False),
)
def _sc_degree(dst_hbm, out_hbm, hist_v, dbuf_v):
    cid = lax.axis_index("c")
    sid = lax.axis_index("s")
    wid = sid * NC + cid

    def zero_body(j, carry):
        hist_v[pl.ds(j * 16, 16)] = jnp.zeros((16,), jnp.float32)
        return carry

    lax.fori_loop(0, NPAD // 16, zero_body, None)

    base = wid * EPW
    ones = jnp.ones((16,), jnp.float32)

    def chunk_body(i, carry):
        pltpu.sync_copy(dst_hbm.at[pl.ds(base + i * DEG_CHUNK, DEG_CHUNK)], dbuf_v)

        def inner(k, c2):
            idx = dbuf_v[pl.ds(k * 16, 16)]
            plsc.addupdate_scatter(hist_v, [idx], ones)
            return c2

        lax.fori_loop(0, DEG_CHUNK // 16, inner, None)
        return carry

    lax.fori_loop(0, EPW // DEG_CHUNK, chunk_body, None)
    pltpu.sync_copy(hist_v, out_hbm.at[wid])


QK = 4
CQ = QK * C
NQ = NCH_TOT // (NC * QK)


@functools.partial(
    pl.kernel,
    out_type=jax.ShapeDtypeStruct((NC, NPAD, D_HID), jnp.float32),
    mesh=_sc_mesh,
    scratch_types=[
        pltpu.VMEM((NQ, CQ), jnp.int32),
        pltpu.VMEM((NQ, CQ), jnp.int32),
        pltpu.VMEM((2, CQ, D_HID), jnp.float32),
        pltpu.VMEM_SHARED((NPAD, D_HID), jnp.float32),
        pltpu.SemaphoreType.DMA,
        pltpu.SemaphoreType.DMA,
        pltpu.SemaphoreType.DMA,
    ],
    compiler_params=pltpu.CompilerParams(
        needs_layout_passes=False, use_tc_tiling_on_sc=False
    ),
)
def _sc_aggregate(y_hbm, src_hbm, dst_hbm, out_hbm, srci, dsti, rows, acc_sh,
                  sem_a, sem_b, sem_s):
    cid = lax.axis_index("c")
    sid = lax.axis_index("s")

    ro = cid * NQ
    pltpu.sync_copy(src_hbm.at[sid, pl.ds(ro, NQ)], srci)
    pltpu.sync_copy(dst_hbm.at[sid, pl.ds(ro, NQ)], dsti)

    r0 = sid * ROWS_PT
    pltpu.sync_copy(y_hbm.at[pl.ds(r0, ROWS_PT)], acc_sh.at[pl.ds(r0, ROWS_PT)])
    plsc.subcore_barrier()

    def fire(g, bufset, sem):
        pltpu.async_copy(y_hbm.at[srci.at[g]], rows.at[bufset], sem)

    def drain_scatter(g, bufset, sem):
        pltpu.make_async_copy(y_hbm.at[pl.ds(0, CQ)], rows.at[bufset], sem).wait()
        pltpu.async_copy(rows.at[bufset], acc_sh.at[dsti.at[g]], sem_s, add=True)
        pltpu.make_async_copy(y_hbm.at[pl.ds(0, CQ)], rows.at[bufset], sem_s).wait()

    fire(0, 0, sem_a)
    fire(1, 1, sem_b)

    def body(t, carry):
        g = 2 * t
        drain_scatter(g, 0, sem_a)
        fire(g + 2, 0, sem_a)
        drain_scatter(g + 1, 1, sem_b)
        fire(g + 3, 1, sem_b)
        return carry

    lax.fori_loop(0, NQ // 2 - 1, body, None)
    drain_scatter(NQ - 2, 0, sem_a)
    drain_scatter(NQ - 1, 1, sem_b)

    plsc.subcore_barrier()
    pltpu.sync_copy(acc_sh.at[pl.ds(r0, ROWS_PT)], out_hbm.at[cid, pl.ds(r0, ROWS_PT)])


ROW_BLK = 2048


def _tc_prep_body(hist_ref, x_ref, w_ref, y_ref, dinv_ref):
    deg = jnp.sum(hist_ref[...], axis=0) + 1.0
    dinv = lax.rsqrt(deg)
    xw = jnp.dot(x_ref[...], w_ref[...], preferred_element_type=jnp.float32)
    y_ref[...] = xw * dinv[:, None]
    dinv_ref[...] = dinv[:, None]


def _tc_prep(hist, xp, W1):
    grid = NPAD // ROW_BLK
    return pl.pallas_call(
        _tc_prep_body,
        grid=(grid,),
        in_specs=[
            pl.BlockSpec((NW, ROW_BLK), lambda i: (0, i)),
            pl.BlockSpec((ROW_BLK, D_IN), lambda i: (i, 0)),
            pl.BlockSpec((D_IN, D_HID), lambda i: (0, 0)),
        ],
        out_specs=[
            pl.BlockSpec((ROW_BLK, D_HID), lambda i: (i, 0)),
            pl.BlockSpec((ROW_BLK, 1), lambda i: (i, 0)),
        ],
        out_shape=[
            jax.ShapeDtypeStruct((NPAD, D_HID), jnp.float32),
            jax.ShapeDtypeStruct((NPAD, 1), jnp.float32),
        ],
    )(hist, xp, W1)


def _tc_mid_body(p_ref, y_ref, dinv_ref, w_ref, b_ref, out_ref):
    dinv = dinv_ref[...]
    s = p_ref[0] + p_ref[1] - y_ref[...]
    h = jnp.maximum(dinv * s + b_ref[...], 0.0)
    out_ref[...] = jnp.dot(h, w_ref[...], preferred_element_type=jnp.float32) * dinv


def _tc_mid(p, y, dinv, W, b):
    grid = NPAD // ROW_BLK
    return pl.pallas_call(
        _tc_mid_body,
        grid=(grid,),
        in_specs=[
            pl.BlockSpec((NC, ROW_BLK, D_HID), lambda i: (0, i, 0)),
            pl.BlockSpec((ROW_BLK, D_HID), lambda i: (i, 0)),
            pl.BlockSpec((ROW_BLK, 1), lambda i: (i, 0)),
            pl.BlockSpec((D_HID, D_HID), lambda i: (0, 0)),
            pl.BlockSpec((1, D_HID), lambda i: (0, 0)),
        ],
        out_specs=pl.BlockSpec((ROW_BLK, D_HID), lambda i: (i, 0)),
        out_shape=jax.ShapeDtypeStruct((NPAD, D_HID), jnp.float32),
    )(p, y, dinv, W, b)


def _tc_mid2_body(p_ref, y_ref, dinv_ref, b_ref, out_ref):
    dinv = dinv_ref[...]
    s = p_ref[0] + p_ref[1] - y_ref[...]
    out_ref[...] = dinv * jnp.maximum(dinv * s + b_ref[...], 0.0)


def _tc_mid2(p, y, dinv, b):
    grid = NPAD // ROW_BLK
    return pl.pallas_call(
        _tc_mid2_body,
        grid=(grid,),
        in_specs=[
            pl.BlockSpec((NC, ROW_BLK, D_HID), lambda i: (0, i, 0)),
            pl.BlockSpec((ROW_BLK, D_HID), lambda i: (i, 0)),
            pl.BlockSpec((ROW_BLK, 1), lambda i: (i, 0)),
            pl.BlockSpec((1, D_HID), lambda i: (0, 0)),
        ],
        out_specs=pl.BlockSpec((ROW_BLK, D_HID), lambda i: (i, 0)),
        out_shape=jax.ShapeDtypeStruct((NPAD, D_HID), jnp.float32),
    )(p, y, dinv, b)


def _tc_fin_body(p_ref, y_ref, dinv_ref, w_ref, b_ref, out_ref):
    dinv = dinv_ref[...]
    z = dinv * (p_ref[0] + p_ref[1] - y_ref[...])
    zw = jnp.dot(z, w_ref[...], preferred_element_type=jnp.float32)
    out_ref[...] = jnp.maximum(zw + b_ref[...], 0.0)


def _tc_fin(p, y, dinv, W, b):
    grid = NPAD // ROW_BLK
    return pl.pallas_call(
        _tc_fin_body,
        grid=(grid,),
        in_specs=[
            pl.BlockSpec((NC, ROW_BLK, D_HID), lambda i: (0, i, 0)),
            pl.BlockSpec((ROW_BLK, D_HID), lambda i: (i, 0)),
            pl.BlockSpec((ROW_BLK, 1), lambda i: (i, 0)),
            pl.BlockSpec((D_HID, D_OUT), lambda i: (0, 0)),
            pl.BlockSpec((1, D_OUT), lambda i: (0, 0)),
        ],
        out_specs=pl.BlockSpec((ROW_BLK, D_OUT), lambda i: (i, 0)),
        out_shape=jax.ShapeDtypeStruct((NPAD, D_OUT), jnp.float32),
    )(p, y, dinv, W, b)


def kernel(x, edge_index, W1, b1, W3, b3, W2, b2):
    pad_idx = jnp.full((EPAD - E,), N, dtype=jnp.int32)
    srcp = jnp.concatenate([edge_index[0], pad_idx])
    dstp = jnp.concatenate([edge_index[1], pad_idx])
    xp = jnp.pad(x, ((0, NPAD - N), (0, 0)))

    src3 = srcp.reshape(NS, NC * NQ, CQ)
    dst3 = dstp.reshape(NS, NC * NQ, CQ)

    hist = _sc_degree(dstp)
    y1, dinv = _tc_prep(hist, xp, W1)
    p1 = _sc_aggregate(y1, src3, dst3)
    y2 = _tc_mid(p1, y1, dinv, W3, b1.reshape(1, -1))
    p2 = _sc_aggregate(y2, src3, dst3)
    y3 = _tc_mid2(p2, y2, dinv, b3.reshape(1, -1))
    p3 = _sc_aggregate(y3, src3, dst3)
    out = _tc_fin(p3, y3, dinv, W2, b2.reshape(1, -1))
    return out[:N]


# speedup vs baseline: 2.2573x; 1.8405x over previous
import functools

import jax
import jax.numpy as jnp
from jax import lax
from jax.experimental import pallas as pl
from jax.experimental.pallas import tpu as pltpu
from jax.experimental.pallas import tpu_sc as plsc

N = 10000
E = 320000
D_IN = 128
D_HID = 64
D_OUT = 128

NC, NS = 2, 16
NW = NC * NS
NPAD = 10240
C = 128
NCH_TOT = 160
EPAD = NS * NCH_TOT * C
EPW = EPAD // NW
ROWS_PT = NPAD // NS

_sc_mesh = plsc.VectorSubcoreMesh(
    core_axis_name="c", subcore_axis_name="s", num_cores=NC, num_subcores=NS
)

DEG_CHUNK = 1024
PH_ROWS = NPAD // 2
TRASH = PH_ROWS
ACC_ROWS = PH_ROWS + 8
CQ = 256
NQP = 22
CAPW = NQP * CQ


@functools.partial(
    pl.kernel,
    out_type=[
        jax.ShapeDtypeStruct((NW, NPAD), jnp.float32),
        jax.ShapeDtypeStruct((NW, 4, CAPW), jnp.int32),
    ],
    mesh=_sc_mesh,
    scratch_types=[
        pltpu.VMEM((NPAD,), jnp.float32),
        pltpu.VMEM((DEG_CHUNK,), jnp.int32),
        pltpu.VMEM((DEG_CHUNK,), jnp.int32),
        pltpu.VMEM((CAPW,), jnp.int32),
        pltpu.VMEM((CAPW,), jnp.int32),
        pltpu.VMEM((CAPW,), jnp.int32),
        pltpu.VMEM((CAPW,), jnp.int32),
    ],
    compiler_params=pltpu.CompilerParams(needs_layout_passes=False),
)
def _sc_prep(src_hbm, dst_hbm, hist_out, lists_out,
             hist_v, sbuf_v, dbuf_v, las_v, lad_v, lbs_v, lbd_v):
    cid = lax.axis_index("c")
    sid = lax.axis_index("s")
    wid = sid * NC + cid

    zero16 = jnp.zeros((16,), jnp.float32)
    pad16 = jnp.full((16,), N, jnp.int32)
    trash16 = jnp.full((16,), TRASH, jnp.int32)

    def zero_body(j, carry):
        hist_v[pl.ds(j * 16, 16)] = zero16
        return carry

    lax.fori_loop(0, NPAD // 16, zero_body, None)

    def fill_body(j, carry):
        las_v[pl.ds(j * 16, 16)] = pad16
        lbs_v[pl.ds(j * 16, 16)] = pad16
        lad_v[pl.ds(j * 16, 16)] = trash16
        lbd_v[pl.ds(j * 16, 16)] = trash16
        return carry

    lax.fori_loop(0, CAPW // 16, fill_body, None)

    base = wid * EPW
    ones = jnp.ones((16,), jnp.float32)
    lim = jnp.int32(CAPW - 16)

    def chunk_body(i, offs):
        pltpu.sync_copy(src_hbm.at[pl.ds(base + i * DEG_CHUNK, DEG_CHUNK)], sbuf_v)
        pltpu.sync_copy(dst_hbm.at[pl.ds(base + i * DEG_CHUNK, DEG_CHUNK)], dbuf_v)

        def inner(k, offs2):
            off_a, off_b = offs2
            s = sbuf_v[pl.ds(k * 16, 16)]
            d = dbuf_v[pl.ds(k * 16, 16)]
            plsc.addupdate_scatter(hist_v, [d], ones)
            m_a = d < PH_ROWS
            m_b = d >= PH_ROWS
            oa = jnp.minimum(off_a, lim)
            ob = jnp.minimum(off_b, lim)
            plsc.store_compressed(las_v.at[pl.ds(oa, 16)], s, mask=m_a)
            plsc.store_compressed(lad_v.at[pl.ds(oa, 16)], d, mask=m_a)
            plsc.store_compressed(lbs_v.at[pl.ds(ob, 16)], s, mask=m_b)
            plsc.store_compressed(lbd_v.at[pl.ds(ob, 16)], d - PH_ROWS, mask=m_b)
            cnt = jnp.sum(m_a.astype(jnp.int32))
            return (off_a + cnt, off_b + (16 - cnt))

        return lax.fori_loop(0, DEG_CHUNK // 16, inner, offs)

    lax.fori_loop(0, EPW // DEG_CHUNK, chunk_body,
                  (jnp.int32(0), jnp.int32(0)))

    pltpu.sync_copy(hist_v, hist_out.at[wid])
    pltpu.sync_copy(las_v, lists_out.at[wid, 0])
    pltpu.sync_copy(lad_v, lists_out.at[wid, 1])
    pltpu.sync_copy(lbs_v, lists_out.at[wid, 2])
    pltpu.sync_copy(lbd_v, lists_out.at[wid, 3])


PH_RPT = PH_ROWS // NS


@functools.partial(
    pl.kernel,
    out_type=jax.ShapeDtypeStruct((NC, NPAD, D_HID), jnp.float32),
    mesh=_sc_mesh,
    scratch_types=[
        pltpu.VMEM((NQP, CQ), jnp.int32),
        pltpu.VMEM((NQP, CQ), jnp.int32),
        pltpu.VMEM((NQP, CQ), jnp.int32),
        pltpu.VMEM((NQP, CQ), jnp.int32),
        pltpu.VMEM((2, CQ, D_HID), jnp.float32),
        pltpu.VMEM_SHARED((NPAD, D_HID), jnp.float32),
        pltpu.VMEM_SHARED((ACC_ROWS, D_HID), jnp.float32),
        pltpu.SemaphoreType.DMA,
        pltpu.SemaphoreType.DMA,
        pltpu.SemaphoreType.DMA,
    ],
    compiler_params=pltpu.CompilerParams(
        needs_layout_passes=False, use_tc_tiling_on_sc=False
    ),
)
def _sc_aggregate(y_hbm, lists_hbm, out_hbm,
                  las_v, lad_v, lbs_v, lbd_v, rows, ytab_sh, acc_sh,
                  sem_a, sem_b, sem_s):
    cid = lax.axis_index("c")
    sid = lax.axis_index("s")
    wid = sid * NC + cid

    pltpu.sync_copy(lists_hbm.at[wid, 0], las_v)
    pltpu.sync_copy(lists_hbm.at[wid, 1], lad_v)
    pltpu.sync_copy(lists_hbm.at[wid, 2], lbs_v)
    pltpu.sync_copy(lists_hbm.at[wid, 3], lbd_v)

    t0 = sid * ROWS_PT
    pltpu.sync_copy(y_hbm.at[pl.ds(t0, ROWS_PT)], ytab_sh.at[pl.ds(t0, ROWS_PT)])

    def fire(srcl, g, bufset, sem):
        pltpu.async_copy(ytab_sh.at[srcl.at[g]], rows.at[bufset], sem)

    def drain_scatter(dstl, g, bufset, sem):
        pltpu.make_async_copy(y_hbm.at[pl.ds(0, CQ)], rows.at[bufset], sem).wait()
        pltpu.async_copy(rows.at[bufset], acc_sh.at[dstl.at[g]], sem_s, add=True)
        pltpu.make_async_copy(y_hbm.at[pl.ds(0, CQ)], rows.at[bufset], sem_s).wait()

    r0 = sid * PH_RPT
    for ph, srcl, dstl in ((0, las_v, lad_v), (1, lbs_v, lbd_v)):
        pltpu.sync_copy(y_hbm.at[pl.ds(ph * PH_ROWS + r0, PH_RPT)],
                        acc_sh.at[pl.ds(r0, PH_RPT)])
        plsc.subcore_barrier()

        fire(srcl, 0, 0, sem_a)
        fire(srcl, 1, 1, sem_b)

        def body(t, carry):
            g = 2 * t
            drain_scatter(dstl, g, 0, sem_a)
            fire(srcl, g + 2, 0, sem_a)
            drain_scatter(dstl, g + 1, 1, sem_b)
            fire(srcl, g + 3, 1, sem_b)
            return carry

        lax.fori_loop(0, NQP // 2 - 1, body, None)
        drain_scatter(dstl, NQP - 2, 0, sem_a)
        drain_scatter(dstl, NQP - 1, 1, sem_b)

        plsc.subcore_barrier()
        pltpu.sync_copy(acc_sh.at[pl.ds(r0, PH_RPT)],
                        out_hbm.at[cid, pl.ds(ph * PH_ROWS + r0, PH_RPT)])


ROW_BLK = 2048


def _tc_prep_body(hist_ref, x_ref, w_ref, y_ref, dinv_ref):
    deg = jnp.sum(hist_ref[...], axis=0) + 1.0
    dinv = lax.rsqrt(deg)
    xw = jnp.dot(x_ref[...], w_ref[...], preferred_element_type=jnp.float32)
    y_ref[...] = xw * dinv[:, None]
    dinv_ref[...] = dinv[:, None]


def _tc_prep(hist, xp, W1):
    grid = NPAD // ROW_BLK
    return pl.pallas_call(
        _tc_prep_body,
        grid=(grid,),
        in_specs=[
            pl.BlockSpec((NW, ROW_BLK), lambda i: (0, i)),
            pl.BlockSpec((ROW_BLK, D_IN), lambda i: (i, 0)),
            pl.BlockSpec((D_IN, D_HID), lambda i: (0, 0)),
        ],
        out_specs=[
            pl.BlockSpec((ROW_BLK, D_HID), lambda i: (i, 0)),
            pl.BlockSpec((ROW_BLK, 1), lambda i: (i, 0)),
        ],
        out_shape=[
            jax.ShapeDtypeStruct((NPAD, D_HID), jnp.float32),
            jax.ShapeDtypeStruct((NPAD, 1), jnp.float32),
        ],
    )(hist, xp, W1)


def _tc_mid_body(p_ref, y_ref, dinv_ref, w_ref, b_ref, out_ref):
    dinv = dinv_ref[...]
    s = p_ref[0] + p_ref[1] - y_ref[...]
    h = jnp.maximum(dinv * s + b_ref[...], 0.0)
    out_ref[...] = jnp.dot(h, w_ref[...], preferred_element_type=jnp.float32) * dinv


def _tc_mid(p, y, dinv, W, b):
    grid = NPAD // ROW_BLK
    return pl.pallas_call(
        _tc_mid_body,
        grid=(grid,),
        in_specs=[
            pl.BlockSpec((NC, ROW_BLK, D_HID), lambda i: (0, i, 0)),
            pl.BlockSpec((ROW_BLK, D_HID), lambda i: (i, 0)),
            pl.BlockSpec((ROW_BLK, 1), lambda i: (i, 0)),
            pl.BlockSpec((D_HID, D_HID), lambda i: (0, 0)),
            pl.BlockSpec((1, D_HID), lambda i: (0, 0)),
        ],
        out_specs=pl.BlockSpec((ROW_BLK, D_HID), lambda i: (i, 0)),
        out_shape=jax.ShapeDtypeStruct((NPAD, D_HID), jnp.float32),
    )(p, y, dinv, W, b)


def _tc_mid2_body(p_ref, y_ref, dinv_ref, b_ref, out_ref):
    dinv = dinv_ref[...]
    s = p_ref[0] + p_ref[1] - y_ref[...]
    out_ref[...] = dinv * jnp.maximum(dinv * s + b_ref[...], 0.0)


def _tc_mid2(p, y, dinv, b):
    grid = NPAD // ROW_BLK
    return pl.pallas_call(
        _tc_mid2_body,
        grid=(grid,),
        in_specs=[
            pl.BlockSpec((NC, ROW_BLK, D_HID), lambda i: (0, i, 0)),
            pl.BlockSpec((ROW_BLK, D_HID), lambda i: (i, 0)),
            pl.BlockSpec((ROW_BLK, 1), lambda i: (i, 0)),
            pl.BlockSpec((1, D_HID), lambda i: (0, 0)),
        ],
        out_specs=pl.BlockSpec((ROW_BLK, D_HID), lambda i: (i, 0)),
        out_shape=jax.ShapeDtypeStruct((NPAD, D_HID), jnp.float32),
    )(p, y, dinv, b)


def _tc_fin_body(p_ref, y_ref, dinv_ref, w_ref, b_ref, out_ref):
    dinv = dinv_ref[...]
    z = dinv * (p_ref[0] + p_ref[1] - y_ref[...])
    zw = jnp.dot(z, w_ref[...], preferred_element_type=jnp.float32)
    out_ref[...] = jnp.maximum(zw + b_ref[...], 0.0)


def _tc_fin(p, y, dinv, W, b):
    grid = NPAD // ROW_BLK
    return pl.pallas_call(
        _tc_fin_body,
        grid=(grid,),
        in_specs=[
            pl.BlockSpec((NC, ROW_BLK, D_HID), lambda i: (0, i, 0)),
            pl.BlockSpec((ROW_BLK, D_HID), lambda i: (i, 0)),
            pl.BlockSpec((ROW_BLK, 1), lambda i: (i, 0)),
            pl.BlockSpec((D_HID, D_OUT), lambda i: (0, 0)),
            pl.BlockSpec((1, D_OUT), lambda i: (0, 0)),
        ],
        out_specs=pl.BlockSpec((ROW_BLK, D_OUT), lambda i: (i, 0)),
        out_shape=jax.ShapeDtypeStruct((NPAD, D_OUT), jnp.float32),
    )(p, y, dinv, W, b)


def kernel(x, edge_index, W1, b1, W3, b3, W2, b2):
    pad_idx = jnp.full((EPAD - E,), N, dtype=jnp.int32)
    srcp = jnp.concatenate([edge_index[0], pad_idx])
    dstp = jnp.concatenate([edge_index[1], pad_idx])
    xp = jnp.pad(x, ((0, NPAD - N), (0, 0)))

    hist, lists = _sc_prep(srcp, dstp)
    lists4 = lists.reshape(NW, 4, NQP, CQ)
    y1, dinv = _tc_prep(hist, xp, W1)
    p1 = _sc_aggregate(y1, lists4)
    y2 = _tc_mid(p1, y1, dinv, W3, b1.reshape(1, -1))
    p2 = _sc_aggregate(y2, lists4)
    y3 = _tc_mid2(p2, y2, dinv, b3.reshape(1, -1))
    p3 = _sc_aggregate(y3, lists4)
    out = _tc_fin(p3, y3, dinv, W2, b2.reshape(1, -1))
    return out[:N]
